# baseline (speedup 1.0000x reference)
import functools

import jax
import jax.numpy as jnp
from jax.experimental import pallas as pl
from jax.experimental.pallas import tpu as pltpu

B, SL, DL, D = 1024, 200, 200, 64
L2 = SL + DL
RB = 8


def _body(src_ref, dst_ref, w1_ref, b1_ref, w2_ref, b2_ref, src_out, dst_out):
    src = src_ref[...]
    dst = dst_ref[...]
    ids = jnp.concatenate([src, dst], axis=1)
    idsf = ids.astype(jnp.float32)
    eq = jnp.where((idsf[:, :, None] == idsf[:, None, :])
                   & (idsf[:, :, None] != 0.0), 1.0, 0.0)
    sel = (jax.lax.broadcasted_iota(jnp.int32, (L2, 2), 0) < SL)
    sel = jnp.where(sel, jnp.float32(1.0), jnp.float32(0.0))
    sel = jnp.where(jax.lax.broadcasted_iota(jnp.int32, (L2, 2), 1) == 0, sel, 1.0 - sel)
    cnt = jnp.dot(eq.reshape(RB * L2, L2), sel,
                  preferred_element_type=jnp.float32)
    w1 = w1_ref[0, :]
    b1 = b1_ref[...]
    h = (jnp.maximum(cnt[:, 0:1] * w1[None, :] + b1, 0.0)
         + jnp.maximum(cnt[:, 1:2] * w1[None, :] + b1, 0.0))
    feat = jnp.dot(h, w2_ref[...], preferred_element_type=jnp.float32)
    feat = feat + 2.0 * b2_ref[...]
    feat = feat.reshape(RB, L2, D)
    src_out[...] = feat[:, :SL, :]
    dst_out[...] = feat[:, SL:, :]


@jax.jit
def kernel(src_ids, dst_ids, W1, b1, W2, b2):
    grid = (B // RB,)
    src_feat, dst_feat = pl.pallas_call(
        _body,
        grid=grid,
        in_specs=[
            pl.BlockSpec((RB, SL), lambda i: (i, 0)),
            pl.BlockSpec((RB, DL), lambda i: (i, 0)),
            pl.BlockSpec((1, D), lambda i: (0, 0)),
            pl.BlockSpec((1, D), lambda i: (0, 0)),
            pl.BlockSpec((D, D), lambda i: (0, 0)),
            pl.BlockSpec((1, D), lambda i: (0, 0)),
        ],
        out_specs=[
            pl.BlockSpec((RB, SL, D), lambda i: (i, 0, 0)),
            pl.BlockSpec((RB, DL, D), lambda i: (i, 0, 0)),
        ],
        out_shape=[
            jax.ShapeDtypeStruct((B, SL, D), jnp.float32),
            jax.ShapeDtypeStruct((B, DL, D), jnp.float32),
        ],
    )(src_ids, dst_ids, W1, b1.reshape(1, D), W2, b2.reshape(1, D))
    return src_feat, dst_feat
